# baseline (speedup 1.0000x reference)
import functools

import jax
import jax.numpy as jnp
from jax import lax
from jax.experimental import pallas as pl
from jax.experimental.pallas import tpu as pltpu
from jax.experimental.pallas import tpu_sc as plsc

VOCAB = 1000
EMB = 32
BATCH = 1024
SEQ = 1000

NC = 2
NS = 16
NW = NC * NS

NBT = BATCH // 128
NSG = NW // NBT
SG = 256

_mesh = plsc.VectorSubcoreMesh(
    core_axis_name="c", subcore_axis_name="s", num_cores=NC, num_subcores=NS
)


@functools.partial(
    pl.kernel,
    out_type=jax.ShapeDtypeStruct((BATCH * SEQ, EMB), jnp.float32),
    mesh=_mesh,
    scratch_types=[
        pltpu.VMEM_SHARED((VOCAB, EMB), jnp.float32),
        pltpu.VMEM((128, SG), jnp.int32),
        pltpu.VMEM((VOCAB,), jnp.int32),
        pltpu.VMEM((128,), jnp.int32),
        pltpu.VMEM((128,), jnp.int32),
        pltpu.VMEM((128, EMB), jnp.float32),
        pltpu.VMEM((128, EMB), jnp.float32),
        pltpu.SemaphoreType.DMA,
        pltpu.SemaphoreType.DMA,
        pltpu.SemaphoreType.DMA,
        pltpu.SemaphoreType.DMA,
    ],
    compiler_params=pltpu.CompilerParams(
        use_tc_tiling_on_sc=False, needs_layout_passes=False
    ),
)
def _emb_gather(
    xp_hbm, w_hbm, mask_hbm, out_hbm, w_sh, xblk, mask_v,
    idx0, idx1, buf0, buf1, gsem0, gsem1, ssem0, ssem1,
):
    sid = lax.axis_index("s")
    wid = sid * NC + lax.axis_index("c")
    bt = wid % NBT
    sgrp = wid // NBT
    s0 = sgrp * SG
    n_s = jnp.where(sgrp == NSG - 1, SEQ - (NSG - 1) * SG, SG)

    @pl.when(sid == 0)
    def _():
        pltpu.sync_copy(w_hbm, w_sh)

    pltpu.sync_copy(mask_hbm, mask_v)
    pltpu.sync_copy(xp_hbm.at[pl.ds(bt * 128, 128), pl.ds(s0, SG)], xblk)
    plsc.subcore_barrier()

    i16 = jnp.arange(16, dtype=jnp.int32)
    z16 = jnp.zeros((16,), jnp.int32)

    def prep_idx(sl, idx_v):
        m16 = plsc.load_gather(mask_v, [z16 + (s0 + sl)])
        for j in range(8):
            idx_v[pl.ds(j * 16, 16)] = (
                plsc.load_gather(xblk, [i16 + j * 16, z16 + sl]) * m16
            )

    def issue_gather(idx_v, buf, gsem):
        pltpu.async_copy(w_sh.at[idx_v], buf, gsem)

    def wait_gather(buf, gsem):
        pltpu.make_async_copy(w_hbm.at[pl.ds(0, 128)], buf, gsem).wait()

    def issue_store(sl, buf, ssem):
        row = (s0 + sl) * BATCH + bt * 128
        pltpu.async_copy(buf, out_hbm.at[pl.ds(row, 128)], ssem)

    def wait_store(buf, ssem):
        pltpu.make_async_copy(buf, out_hbm.at[pl.ds(0, 128)], ssem).wait()

    prep_idx(0, idx0)
    issue_gather(idx0, buf0, gsem0)
    prep_idx(1, idx1)
    issue_gather(idx1, buf1, gsem1)

    @pl.loop(0, n_s // 2 - 1)
    def _step(i):
        sl = 2 * i
        wait_gather(buf0, gsem0)
        issue_store(sl, buf0, ssem0)
        prep_idx(sl + 2, idx0)
        wait_store(buf0, ssem0)
        issue_gather(idx0, buf0, gsem0)
        wait_gather(buf1, gsem1)
        issue_store(sl + 1, buf1, ssem1)
        prep_idx(sl + 3, idx1)
        wait_store(buf1, ssem1)
        issue_gather(idx1, buf1, gsem1)

    wait_gather(buf0, gsem0)
    pltpu.sync_copy(buf0, out_hbm.at[pl.ds((s0 + n_s - 2) * BATCH + bt * 128, 128)])
    wait_gather(buf1, gsem1)
    pltpu.sync_copy(buf1, out_hbm.at[pl.ds((s0 + n_s - 1) * BATCH + bt * 128, 128)])


_TCG = 8


def _tc_transpose_body(in_ref, out_ref):
    for sg in range(_TCG):
        for bt in range(NBT):
            c = in_ref[pl.ds((sg * NBT + bt) * 32, 32), :]
            c = c.reshape(128, EMB).T
            out_ref[sg, :, bt, :, :] = c.reshape(4, 8, 128)


_tc_transpose = pl.pallas_call(
    _tc_transpose_body,
    grid=(SEQ // _TCG,),
    in_specs=[pl.BlockSpec((_TCG * 256, 128), lambda i: (i, 0))],
    out_specs=pl.BlockSpec(
        (_TCG, 4, NBT, 8, 128), lambda i: (i, 0, 0, 0, 0)
    ),
    out_shape=jax.ShapeDtypeStruct((SEQ, 4, NBT, 8, 128), jnp.float32),
)


def kernel(x, W, mask):
    xp = jnp.pad(x, ((0, 0), (0, SG * NSG - SEQ)))
    sbe = _emb_gather(xp, W, mask.reshape(-1).astype(jnp.int32))
    b = _tc_transpose(sbe.reshape(SEQ * 256, 128))
    return b.transpose(2, 4, 0, 1, 3).reshape(BATCH, SEQ, EMB)


# speedup vs baseline: 1.2027x; 1.2027x over previous
import functools

import jax
import jax.numpy as jnp
from jax import lax
from jax.experimental import pallas as pl
from jax.experimental.pallas import tpu as pltpu
from jax.experimental.pallas import tpu_sc as plsc

VOCAB = 1000
EMB = 32
BATCH = 1024
SEQ = 1000

NC = 2
NS = 16
NW = NC * NS

NBT = BATCH // 128
NSG = NW // NBT
SG = 256

_mesh = plsc.VectorSubcoreMesh(
    core_axis_name="c", subcore_axis_name="s", num_cores=NC, num_subcores=NS
)


@functools.partial(
    pl.kernel,
    out_type=jax.ShapeDtypeStruct((SEQ * 256, 128), jnp.float32),
    mesh=_mesh,
    scratch_types=[
        pltpu.VMEM((EMB, VOCAB), jnp.float32),
        pltpu.VMEM((128, SG), jnp.int32),
        pltpu.VMEM((VOCAB,), jnp.int32),
        pltpu.VMEM((EMB, 128), jnp.float32),
        pltpu.VMEM((EMB, 128), jnp.float32),
        pltpu.SemaphoreType.DMA,
        pltpu.SemaphoreType.DMA,
    ],
    compiler_params=pltpu.CompilerParams(
        use_tc_tiling_on_sc=False, needs_layout_passes=False
    ),
)
def _emb_lookup(
    xp_hbm, wt_hbm, mask_hbm, out_hbm, wt_v, xblk, mask_v,
    buf0, buf1, ssem0, ssem1,
):
    wid = lax.axis_index("s") * NC + lax.axis_index("c")
    bt = wid % NBT
    sgrp = wid // NBT
    s0 = sgrp * SG
    n_s = jnp.where(sgrp == NSG - 1, SEQ - (NSG - 1) * SG, SG)

    pltpu.sync_copy(wt_hbm, wt_v)
    pltpu.sync_copy(mask_hbm, mask_v)
    pltpu.sync_copy(
        xp_hbm.at[pl.ds(bt * 128, 128), pl.ds(s0, SG)], xblk
    )

    i16 = jnp.arange(16, dtype=jnp.int32)
    z16 = jnp.zeros((16,), jnp.int32)

    def compute(sl, buf):
        s = s0 + sl
        m16 = plsc.load_gather(mask_v, [z16 + s])
        for j in range(8):
            idx = plsc.load_gather(xblk, [i16 + j * 16, z16 + sl]) * m16
            for e in range(EMB):
                buf[e, pl.ds(j * 16, 16)] = plsc.load_gather(wt_v, [z16 + e, idx])

    def issue_stores(sl, buf, ssem):
        s = s0 + sl
        for t in range(4):
            pltpu.async_copy(
                buf.at[pl.ds(t * 8, 8)],
                out_hbm.at[pl.ds(s * 256 + t * 64 + bt * 8, 8)],
                ssem,
            )

    def wait_stores(buf, ssem):
        pltpu.make_async_copy(buf, out_hbm.at[pl.ds(0, EMB)], ssem).wait()

    compute(0, buf0)
    issue_stores(0, buf0, ssem0)
    compute(1, buf1)
    issue_stores(1, buf1, ssem1)

    @pl.loop(0, n_s // 2 - 1)
    def _step(i):
        sl = 2 * i
        wait_stores(buf0, ssem0)
        compute(sl + 2, buf0)
        issue_stores(sl + 2, buf0, ssem0)
        wait_stores(buf1, ssem1)
        compute(sl + 3, buf1)
        issue_stores(sl + 3, buf1, ssem1)

    wait_stores(buf0, ssem0)
    wait_stores(buf1, ssem1)


def kernel(x, W, mask):
    xp = jnp.pad(x, ((0, 0), (0, SG * NSG - SEQ)))
    b = _emb_lookup(xp, W.T, mask.reshape(-1).astype(jnp.int32))
    return (
        b.reshape(SEQ, 4, 8, 8, 128)
        .transpose(2, 4, 0, 1, 3)
        .reshape(BATCH, SEQ, EMB)
    )
